# baseline (speedup 1.0000x reference)
import jax
import jax.numpy as jnp
from jax.experimental import pallas as pl
from jax.experimental.pallas import tpu as pltpu

_B, _J, _H, _W = 128, 17, 64, 48
_HW = _H * _W
_ROWS = _J * _HW
_GRID = 12
_RB = _ROWS // _GRID
_SEG = _RB // _J
_SPJ = _HW // _SEG
_NSEG = _GRID * _J
_TOPK = 8


def _ohkm_kernel(o_ref, t_ref, w_ref, out_ref, s_ref):
    i = pl.program_id(0)
    d = o_ref[...] - t_ref[...]
    part = jnp.sum((d * d).reshape(_J, _SEG, _B), axis=1)
    s_ref[pl.ds(i, 1)] = part[None]

    @pl.when(i == _GRID - 1)
    def _():
        rows = []
        for j in range(_J):
            g0 = j * _SPJ
            r = s_ref[g0 // _J, g0 % _J, :]
            for m in range(1, _SPJ):
                g = g0 + m
                r = r + s_ref[g // _J, g % _J, :]
            rows.append(r)
        s = jnp.stack(rows, axis=0)
        w = w_ref[...]
        vals = s * (w * w) * (0.5 / _HW)
        row = jax.lax.broadcasted_iota(jnp.int32, vals.shape, 0)
        acc = jnp.zeros((_B,), jnp.float32)
        neg_inf = jnp.float32(-jnp.inf)
        for _ in range(_TOPK):
            m = jnp.max(vals, axis=0)
            acc = acc + m
            is_max = vals == m[None, :]
            first_idx = jnp.min(jnp.where(is_max, row, _J), axis=0)
            vals = jnp.where(row == first_idx[None, :], neg_inf, vals)
        out_ref[0, 0] = jnp.sum(acc) * (1.0 / (_TOPK * _B))


def kernel(outs, targets, target_weights):
    o = jnp.transpose(outs, (1, 2, 3, 0)).reshape(_ROWS, _B)
    t = jnp.transpose(targets, (1, 2, 3, 0)).reshape(_ROWS, _B)
    w = jnp.transpose(target_weights, (1, 2, 0)).reshape(_J, _B)
    out = pl.pallas_call(
        _ohkm_kernel,
        grid=(_GRID,),
        in_specs=[
            pl.BlockSpec((_RB, _B), lambda i: (i, 0)),
            pl.BlockSpec((_RB, _B), lambda i: (i, 0)),
            pl.BlockSpec((_J, _B), lambda i: (0, 0)),
        ],
        out_specs=pl.BlockSpec(
            (1, 1), lambda i: (0, 0), memory_space=pltpu.SMEM
        ),
        out_shape=jax.ShapeDtypeStruct((1, 1), jnp.float32),
        scratch_shapes=[pltpu.VMEM((_GRID, _J, _B), jnp.float32)],
    )(o, t, w)
    return out.reshape(())


# speedup vs baseline: 1.0589x; 1.0589x over previous
import jax
import jax.numpy as jnp
from jax.experimental import pallas as pl
from jax.experimental.pallas import tpu as pltpu

_B, _J, _H, _W = 128, 17, 64, 48
_HW = _H * _W
_ROWS = _J * _HW
_GRID = 6
_RB = _ROWS // _GRID
_SEG = _RB // _J
_SPJ = _HW // _SEG
_NSEG = _GRID * _J
_TOPK = 8


def _ohkm_kernel(o_ref, t_ref, w_ref, out_ref, s_ref):
    i = pl.program_id(0)
    segs = []
    for k in range(_J):
        dk = o_ref[k * _SEG:(k + 1) * _SEG, :] - t_ref[k * _SEG:(k + 1) * _SEG, :]
        segs.append(jnp.sum(dk * dk, axis=0))
    part = jnp.stack(segs, axis=0)
    s_ref[pl.ds(i, 1)] = part[None]

    @pl.when(i == _GRID - 1)
    def _():
        rows = []
        for j in range(_J):
            g0 = j * _SPJ
            r = s_ref[g0 // _J, g0 % _J, :]
            for m in range(1, _SPJ):
                g = g0 + m
                r = r + s_ref[g // _J, g % _J, :]
            rows.append(r)
        s = jnp.stack(rows, axis=0)
        w = w_ref[...]
        vals = s * (w * w) * (0.5 / _HW)
        row = jax.lax.broadcasted_iota(jnp.int32, vals.shape, 0)
        acc = jnp.zeros((_B,), jnp.float32)
        neg_inf = jnp.float32(-jnp.inf)
        for _ in range(_TOPK):
            m = jnp.max(vals, axis=0)
            acc = acc + m
            is_max = vals == m[None, :]
            first_idx = jnp.min(jnp.where(is_max, row, _J), axis=0)
            vals = jnp.where(row == first_idx[None, :], neg_inf, vals)
        out_ref[0, 0] = jnp.sum(acc) * (1.0 / (_TOPK * _B))


def kernel(outs, targets, target_weights):
    o = jnp.transpose(outs, (1, 2, 3, 0)).reshape(_ROWS, _B)
    t = jnp.transpose(targets, (1, 2, 3, 0)).reshape(_ROWS, _B)
    w = jnp.transpose(target_weights, (1, 2, 0)).reshape(_J, _B)
    out = pl.pallas_call(
        _ohkm_kernel,
        grid=(_GRID,),
        in_specs=[
            pl.BlockSpec((_RB, _B), lambda i: (i, 0)),
            pl.BlockSpec((_RB, _B), lambda i: (i, 0)),
            pl.BlockSpec((_J, _B), lambda i: (0, 0)),
        ],
        out_specs=pl.BlockSpec(
            (1, 1), lambda i: (0, 0), memory_space=pltpu.SMEM
        ),
        out_shape=jax.ShapeDtypeStruct((1, 1), jnp.float32),
        scratch_shapes=[pltpu.VMEM((_GRID, _J, _B), jnp.float32)],
    )(o, t, w)
    return out.reshape(())


# speedup vs baseline: 1.0606x; 1.0016x over previous
import jax
import jax.numpy as jnp
from jax.experimental import pallas as pl
from jax.experimental.pallas import tpu as pltpu

_B, _J, _H, _W = 128, 17, 64, 48
_HW = _H * _W
_ROWS = _J * _HW
_GRID = 3
_RB = _ROWS // _GRID
_SEG = _RB // _J
_SPJ = _HW // _SEG
_NSEG = _GRID * _J
_TOPK = 8


def _ohkm_kernel(o_ref, t_ref, w_ref, out_ref, s_ref):
    i = pl.program_id(0)
    segs = []
    for k in range(_J):
        dk = o_ref[k * _SEG:(k + 1) * _SEG, :] - t_ref[k * _SEG:(k + 1) * _SEG, :]
        segs.append(jnp.sum(dk * dk, axis=0))
    part = jnp.stack(segs, axis=0)
    s_ref[pl.ds(i, 1)] = part[None]

    @pl.when(i == _GRID - 1)
    def _():
        rows = []
        for j in range(_J):
            g0 = j * _SPJ
            r = s_ref[g0 // _J, g0 % _J, :]
            for m in range(1, _SPJ):
                g = g0 + m
                r = r + s_ref[g // _J, g % _J, :]
            rows.append(r)
        s = jnp.stack(rows, axis=0)
        w = w_ref[...]
        vals = s * (w * w) * (0.5 / _HW)
        row = jax.lax.broadcasted_iota(jnp.int32, vals.shape, 0)
        acc = jnp.zeros((_B,), jnp.float32)
        neg_inf = jnp.float32(-jnp.inf)
        for _ in range(_TOPK):
            m = jnp.max(vals, axis=0)
            acc = acc + m
            is_max = vals == m[None, :]
            first_idx = jnp.min(jnp.where(is_max, row, _J), axis=0)
            vals = jnp.where(row == first_idx[None, :], neg_inf, vals)
        out_ref[0, 0] = jnp.sum(acc) * (1.0 / (_TOPK * _B))


def kernel(outs, targets, target_weights):
    o = jnp.transpose(outs, (1, 2, 3, 0)).reshape(_ROWS, _B)
    t = jnp.transpose(targets, (1, 2, 3, 0)).reshape(_ROWS, _B)
    w = jnp.transpose(target_weights, (1, 2, 0)).reshape(_J, _B)
    out = pl.pallas_call(
        _ohkm_kernel,
        grid=(_GRID,),
        in_specs=[
            pl.BlockSpec((_RB, _B), lambda i: (i, 0)),
            pl.BlockSpec((_RB, _B), lambda i: (i, 0)),
            pl.BlockSpec((_J, _B), lambda i: (0, 0)),
        ],
        out_specs=pl.BlockSpec(
            (1, 1), lambda i: (0, 0), memory_space=pltpu.SMEM
        ),
        out_shape=jax.ShapeDtypeStruct((1, 1), jnp.float32),
        scratch_shapes=[pltpu.VMEM((_GRID, _J, _B), jnp.float32)],
    )(o, t, w)
    return out.reshape(())


# speedup vs baseline: 1.0607x; 1.0001x over previous
import jax
import jax.numpy as jnp
from jax.experimental import pallas as pl
from jax.experimental.pallas import tpu as pltpu

_B, _J, _H, _W = 128, 17, 64, 48
_HW = _H * _W
_RB = 512
_GRID = _HW // _RB
_TOPK = 8


def _ohkm_kernel(o_ref, t_ref, w_ref, out_ref, s_ref):
    i = pl.program_id(0)
    d = o_ref[...] - t_ref[...]
    part = jnp.sum(d * d, axis=1)

    @pl.when(i == 0)
    def _():
        s_ref[...] = jnp.zeros((_J, _B), jnp.float32)

    s_ref[...] += part

    @pl.when(i == _GRID - 1)
    def _():
        w = w_ref[...]
        vals = s_ref[...] * (w * w) * (0.5 / _HW)
        row = jax.lax.broadcasted_iota(jnp.int32, vals.shape, 0)
        acc = jnp.zeros((_B,), jnp.float32)
        neg_inf = jnp.float32(-jnp.inf)
        for _ in range(_TOPK):
            m = jnp.max(vals, axis=0)
            acc = acc + m
            is_max = vals == m[None, :]
            first_idx = jnp.min(jnp.where(is_max, row, _J), axis=0)
            vals = jnp.where(row == first_idx[None, :], neg_inf, vals)
        out_ref[0, 0] = jnp.sum(acc) * (1.0 / (_TOPK * _B))


def kernel(outs, targets, target_weights):
    o = jnp.transpose(outs, (1, 2, 3, 0)).reshape(_J, _HW, _B)
    t = jnp.transpose(targets, (1, 2, 3, 0)).reshape(_J, _HW, _B)
    w = jnp.transpose(target_weights, (1, 2, 0)).reshape(_J, _B)
    out = pl.pallas_call(
        _ohkm_kernel,
        grid=(_GRID,),
        in_specs=[
            pl.BlockSpec((_J, _RB, _B), lambda i: (0, i, 0)),
            pl.BlockSpec((_J, _RB, _B), lambda i: (0, i, 0)),
            pl.BlockSpec((_J, _B), lambda i: (0, 0)),
        ],
        out_specs=pl.BlockSpec(
            (1, 1), lambda i: (0, 0), memory_space=pltpu.SMEM
        ),
        out_shape=jax.ShapeDtypeStruct((1, 1), jnp.float32),
        scratch_shapes=[pltpu.VMEM((_J, _B), jnp.float32)],
    )(o, t, w)
    return out.reshape(())
